# baseline (speedup 1.0000x reference)
import functools

import jax
import jax.numpy as jnp
from jax import lax
from jax.experimental import pallas as pl
from jax.experimental.pallas import tpu as pltpu
from jax.experimental.pallas import tpu_sc as plsc

N = 10000
E = 160000
D = 256
HALF = 128
LANES = 16
C = 96
NB = 3
TILES = 16
NPAD = 10240
RPT = NPAD // TILES
LASTR = N - 15 * RPT
EPAD = 165888
EPT = EPAD // TILES
NCHUNK = EPT // C


def _scores_body(x_ref, w_ref, be_ref, wl_ref, bl_ref, wn_ref, bn_ref,
                 o_ref, x2_ref):
    w = w_ref[...]
    wl = wl_ref[...]
    wn = wn_ref[...]
    wv = jnp.concatenate([wl, wn], axis=0)
    uv = jnp.dot(wv, w, preferred_element_type=jnp.float32)
    be = be_ref[...]
    consts = jnp.sum(wv * be, axis=1)[None, :]
    consts = consts + jnp.concatenate([bl_ref[...], bn_ref[...]], axis=1)
    xv = x_ref[...]
    y = jnp.dot(xv, uv.T, preferred_element_type=jnp.float32)
    o_ref[...] = y + consts
    x2_ref[pl.ds(0, N), :] = xv[:, :HALF]
    x2_ref[pl.ds(N, N), :] = xv[:, HALF:]


def _scores_tc(x, W_emb, b_emb, w_local, b_local, w_nbr, b_nbr):
    return pl.pallas_call(
        _scores_body,
        out_shape=[jax.ShapeDtypeStruct((N, 2), jnp.float32),
                   jax.ShapeDtypeStruct((2 * N, HALF), jnp.float32)],
    )(x, W_emb, b_emb, w_local, b_local, w_nbr, b_nbr)


@functools.partial(
    pl.kernel,
    out_type=jax.ShapeDtypeStruct((N, D), jnp.float32),
    mesh=plsc.VectorSubcoreMesh(core_axis_name="c", subcore_axis_name="s"),
    compiler_params=pltpu.CompilerParams(needs_layout_passes=False),
    scratch_types=[
        pltpu.VMEM((EPT,), jnp.float32),
        [pltpu.VMEM((2 * C,), jnp.int32)] * NB,
        [pltpu.SemaphoreType.DMA] * NB,
        [pltpu.SemaphoreType.DMA] * NB,
        [pltpu.SemaphoreType.DMA] * NB,
        pltpu.VMEM_SHARED((NPAD, HALF), jnp.float32),
    ],
)
def _edge_sc(x2_h, local_h, nbr_h, epk_h, out_h,
             scores_all, eidx_v, sems, ssems, isems, acc):
    cid = lax.axis_index("c")
    sid = lax.axis_index("s")
    off = cid * N

    def load_idx(g, b):
        base = pl.multiple_of((sid * NCHUNK + g) * 2 * C, 8)
        pltpu.async_copy(epk_h.at[pl.ds(base, 2 * C)], eidx_v[b], isems[b])

    def wait_idx(g, b):
        base = pl.multiple_of((sid * NCHUNK + g) * 2 * C, 8)
        pltpu.make_async_copy(epk_h.at[pl.ds(base, 2 * C)], eidx_v[b],
                              isems[b]).wait()

    def phase0(local_v, nbr_v):
        pltpu.sync_copy(local_h, local_v)
        pltpu.sync_copy(nbr_h, nbr_v)
        load_idx(0, 0)
        load_idx(1, 1)
        load_idx(2, 2)

        def p0_body(k, carry):
            for b in range(NB):
                g = NB * k + b
                wait_idx(g, b)
                for i in range(C // LANES):
                    sl = pl.ds(i * LANES, LANES)
                    sv = eidx_v[b][sl]
                    dv = eidx_v[b][pl.ds(C + i * LANES, LANES)]
                    scores_all[pl.ds(g * C + i * LANES, LANES)] = (
                        plsc.load_gather(local_v, [dv])
                        + plsc.load_gather(nbr_v, [sv]))

                @pl.when(g + 3 < NCHUNK)
                def _():
                    load_idx(g + 3, b)

            return carry

        lax.fori_loop(0, NCHUNK // NB, p0_body, 0)

    pl.run_scoped(phase0,
                  pltpu.VMEM((NPAD,), jnp.float32),
                  pltpu.VMEM((NPAD,), jnp.float32))

    def phase1(rows_v, dstg_v, gidx_v):
        zb = rows_v[0]

        def zero_body(i, c2):
            for j in range(HALF // LANES):
                zb[i, pl.ds(j * LANES, LANES)] = jnp.zeros((LANES,),
                                                           jnp.float32)
            return c2

        lax.fori_loop(0, C, zero_body, 0)
        rbase = pl.multiple_of(sid * RPT, 8)
        for t in range(6):
            pltpu.sync_copy(zb, acc.at[pl.ds(rbase + t * C, C)])
        pltpu.sync_copy(zb.at[pl.ds(0, RPT - 6 * C)],
                        acc.at[pl.ds(rbase + 6 * C, RPT - 6 * C)])
        plsc.subcore_barrier()

        def prep(g, b):
            wait_idx(g, b)
            for i in range(C // LANES):
                sl = pl.ds(i * LANES, LANES)
                gidx_v[b][sl] = eidx_v[b][sl] + off
                dstg_v[b][sl] = eidx_v[b][pl.ds(C + i * LANES, LANES)]
            pltpu.async_copy(x2_h.at[gidx_v[b]], rows_v[b], sems[b])

        load_idx(0, 0)
        load_idx(1, 1)
        load_idx(2, 2)
        prep(0, 0)
        load_idx(3, 0)
        prep(1, 1)
        load_idx(4, 1)

        def triple_body(k, carry):
            for b in range(NB):
                g = NB * k + b
                b2 = (b + 2) % NB
                pltpu.make_async_copy(x2_h.at[gidx_v[b]], rows_v[b],
                                      sems[b]).wait()

                def scale_body(i, c2):
                    svec = scores_all[pl.ds(g * C + i * LANES, LANES)]
                    for kk in range(LANES):
                        s = svec[kk]
                        e = i * LANES + kk
                        for j in range(HALF // LANES):
                            sl2 = pl.ds(j * LANES, LANES)
                            rows_v[b][e, sl2] = rows_v[b][e, sl2] * s
                    return c2

                lax.fori_loop(0, C // LANES, scale_body, 0)
                pltpu.async_copy(rows_v[b], acc.at[dstg_v[b]], ssems[b],
                                 add=True)

                @pl.when(g + 2 < NCHUNK)
                def _():
                    @pl.when(g >= 1)
                    def _():
                        pltpu.make_async_copy(rows_v[b2], acc.at[dstg_v[b2]],
                                              ssems[b2]).wait()

                    prep(g + 2, b2)

                @pl.when(g + 5 < NCHUNK)
                def _():
                    load_idx(g + 5, b2)

            return carry

        lax.fori_loop(0, NCHUNK // NB, triple_body, 0)
        for b in range(NB):
            pltpu.make_async_copy(rows_v[b], acc.at[dstg_v[b]],
                                  ssems[b]).wait()
        plsc.subcore_barrier()
        cbase = pl.multiple_of(cid * HALF, HALF)

        @pl.when(sid < TILES - 1)
        def _():
            pltpu.sync_copy(acc.at[pl.ds(rbase, RPT)],
                            out_h.at[pl.ds(rbase, RPT), pl.ds(cbase, HALF)])

        @pl.when(sid == TILES - 1)
        def _():
            pltpu.sync_copy(acc.at[pl.ds(rbase, LASTR)],
                            out_h.at[pl.ds(rbase, LASTR), pl.ds(cbase, HALF)])

    pl.run_scoped(phase1,
                  [pltpu.VMEM((C, HALF), jnp.float32)] * NB,
                  [pltpu.VMEM((C,), jnp.int32)] * NB,
                  [pltpu.VMEM((C,), jnp.int32)] * NB)


def kernel(x, edge_index, W_emb, b_emb, w_local, b_local, w_nbr, b_nbr):
    src = edge_index[0].astype(jnp.int32)
    dst = edge_index[1].astype(jnp.int32)
    ln, x2 = _scores_tc(x, W_emb, b_emb[None, :], w_local, b_local[None, :],
                        w_nbr, b_nbr[None, :])
    pad_n = jnp.zeros((NPAD - N,), jnp.float32)
    local = jnp.concatenate([ln[:, 0], pad_n])
    nbr = jnp.concatenate([ln[:, 1], pad_n])
    src = jnp.concatenate([src, jnp.zeros((EPAD - E,), jnp.int32)])
    dst = jnp.concatenate([dst, jnp.full((EPAD - E,), N, jnp.int32)])
    epk = jnp.stack([src.reshape(TILES, NCHUNK, C),
                     dst.reshape(TILES, NCHUNK, C)], axis=2).reshape(-1)
    return _edge_sc(x2, local, nbr, epk)


# speedup vs baseline: 1.3800x; 1.3800x over previous
import functools

import jax
import jax.numpy as jnp
from jax import lax
from jax.experimental import pallas as pl
from jax.experimental.pallas import tpu as pltpu
from jax.experimental.pallas import tpu_sc as plsc

N = 10000
E = 160000
D = 256
HALF = 128
LANES = 16
C = 64
NB = 3
TILES = 16
NPAD = 10240
RPT = NPAD // TILES
LASTR = N - 15 * RPT
EPAD = 162816
EPT = EPAD // TILES
NCHUNK = EPT // C


def _scores_body(x_ref, w_ref, be_ref, wl_ref, bl_ref, wn_ref, bn_ref,
                 o_ref, x2_ref):
    w = w_ref[...]
    wl = wl_ref[...]
    wn = wn_ref[...]
    wv = jnp.concatenate([wl, wn], axis=0)
    uv = jnp.dot(wv, w, preferred_element_type=jnp.float32)
    be = be_ref[...]
    consts = jnp.sum(wv * be, axis=1)[None, :]
    consts = consts + jnp.concatenate([bl_ref[...], bn_ref[...]], axis=1)
    xv = x_ref[...]
    y = jnp.dot(xv, uv.T, preferred_element_type=jnp.float32)
    o_ref[...] = y + consts
    x2_ref[pl.ds(0, N), :] = xv[:, :HALF]
    x2_ref[pl.ds(N, N), :] = xv[:, HALF:]


def _scores_tc(x, W_emb, b_emb, w_local, b_local, w_nbr, b_nbr):
    return pl.pallas_call(
        _scores_body,
        out_shape=[jax.ShapeDtypeStruct((N, 2), jnp.float32),
                   jax.ShapeDtypeStruct((2 * N, HALF), jnp.float32)],
    )(x, W_emb, b_emb, w_local, b_local, w_nbr, b_nbr)


@functools.partial(
    pl.kernel,
    out_type=jax.ShapeDtypeStruct((N, D), jnp.float32),
    mesh=plsc.VectorSubcoreMesh(core_axis_name="c", subcore_axis_name="s"),
    compiler_params=pltpu.CompilerParams(needs_layout_passes=False),
    scratch_types=[
        pltpu.VMEM((NPAD,), jnp.float32),
        pltpu.VMEM((NPAD,), jnp.float32),
        [pltpu.VMEM((2 * C,), jnp.int32)] * NB,
        [pltpu.VMEM((C,), jnp.int32)] * NB,
        [pltpu.VMEM((C,), jnp.int32)] * NB,
        [pltpu.VMEM((C,), jnp.float32)] * NB,
        [pltpu.VMEM((C, HALF), jnp.float32)] * NB,
        [pltpu.SemaphoreType.DMA] * NB,
        [pltpu.SemaphoreType.DMA] * NB,
        [pltpu.SemaphoreType.DMA] * NB,
        pltpu.VMEM_SHARED((NPAD, HALF), jnp.float32),
    ],
)
def _edge_sc(x2_h, local_h, nbr_h, epk_h, out_h,
             local_v, nbr_v, eidx_v, dstg_v, gidx_v, scores_v, rows_v, sems,
             ssems, isems, acc):
    cid = lax.axis_index("c")
    sid = lax.axis_index("s")

    pltpu.sync_copy(local_h, local_v)
    pltpu.sync_copy(nbr_h, nbr_v)
    zb = rows_v[0]

    def zero_body(i, c2):
        for j in range(HALF // LANES):
            zb[i, pl.ds(j * LANES, LANES)] = jnp.zeros((LANES,), jnp.float32)
        return c2

    lax.fori_loop(0, C, zero_body, 0)
    rbase = pl.multiple_of(sid * RPT, 8)
    for t in range(RPT // C):
        pltpu.sync_copy(zb, acc.at[pl.ds(rbase + t * C, C)])
    plsc.subcore_barrier()

    off = cid * N

    def load_idx(g, b):
        base = pl.multiple_of((sid * NCHUNK + g) * 2 * C, 8)
        pltpu.async_copy(epk_h.at[pl.ds(base, 2 * C)], eidx_v[b], isems[b])

    def prep(g, b):
        base = pl.multiple_of((sid * NCHUNK + g) * 2 * C, 8)
        pltpu.make_async_copy(epk_h.at[pl.ds(base, 2 * C)], eidx_v[b],
                              isems[b]).wait()
        for i in range(C // LANES):
            sl = pl.ds(i * LANES, LANES)
            sv = eidx_v[b][sl]
            dv = eidx_v[b][pl.ds(C + i * LANES, LANES)]
            gidx_v[b][sl] = sv + off
            dstg_v[b][sl] = dv
            scores_v[b][sl] = (plsc.load_gather(local_v, [dv])
                               + plsc.load_gather(nbr_v, [sv]))
        pltpu.async_copy(x2_h.at[gidx_v[b]], rows_v[b], sems[b])

    load_idx(0, 0)
    load_idx(1, 1)
    load_idx(2, 2)
    prep(0, 0)
    load_idx(3, 0)
    prep(1, 1)
    load_idx(4, 1)

    def triple_body(k, carry):
        for b in range(NB):
            g = NB * k + b
            b2 = (b + 2) % NB
            pltpu.make_async_copy(x2_h.at[gidx_v[b]], rows_v[b],
                                  sems[b]).wait()

            def scale_body(i, c2):
                svec = scores_v[b][pl.ds(i * LANES, LANES)]
                for kk in range(LANES):
                    s = svec[kk]
                    e = i * LANES + kk
                    for j in range(HALF // LANES):
                        sl2 = pl.ds(j * LANES, LANES)
                        rows_v[b][e, sl2] = rows_v[b][e, sl2] * s
                return c2

            lax.fori_loop(0, C // LANES, scale_body, 0, unroll=True)
            pltpu.async_copy(rows_v[b], acc.at[dstg_v[b]], ssems[b], add=True)

            @pl.when(g + 2 < NCHUNK)
            def _():
                @pl.when(g >= 1)
                def _():
                    pltpu.make_async_copy(rows_v[b2], acc.at[dstg_v[b2]],
                                          ssems[b2]).wait()

                prep(g + 2, b2)

            @pl.when(g + 5 < NCHUNK)
            def _():
                load_idx(g + 5, b2)

        return carry

    lax.fori_loop(0, NCHUNK // NB, triple_body, 0)
    for b in range(NB):
        pltpu.make_async_copy(rows_v[b], acc.at[dstg_v[b]], ssems[b]).wait()
    plsc.subcore_barrier()
    cbase = pl.multiple_of(cid * HALF, HALF)

    @pl.when(sid < TILES - 1)
    def _():
        pltpu.sync_copy(acc.at[pl.ds(rbase, RPT)],
                        out_h.at[pl.ds(rbase, RPT), pl.ds(cbase, HALF)])

    @pl.when(sid == TILES - 1)
    def _():
        pltpu.sync_copy(acc.at[pl.ds(rbase, LASTR)],
                        out_h.at[pl.ds(rbase, LASTR), pl.ds(cbase, HALF)])


def kernel(x, edge_index, W_emb, b_emb, w_local, b_local, w_nbr, b_nbr):
    src = edge_index[0].astype(jnp.int32)
    dst = edge_index[1].astype(jnp.int32)
    ln, x2 = _scores_tc(x, W_emb, b_emb[None, :], w_local, b_local[None, :],
                        w_nbr, b_nbr[None, :])
    pad_n = jnp.zeros((NPAD - N,), jnp.float32)
    local = jnp.concatenate([ln[:, 0], pad_n])
    nbr = jnp.concatenate([ln[:, 1], pad_n])
    src = jnp.concatenate([src, jnp.zeros((EPAD - E,), jnp.int32)])
    dst = jnp.concatenate([dst, jnp.full((EPAD - E,), N, jnp.int32)])
    epk = jnp.stack([src.reshape(TILES, NCHUNK, C),
                     dst.reshape(TILES, NCHUNK, C)], axis=2).reshape(-1)
    return _edge_sc(x2, local, nbr, epk)


# speedup vs baseline: 1.4618x; 1.0592x over previous
import functools

import jax
import jax.numpy as jnp
from jax import lax
from jax.experimental import pallas as pl
from jax.experimental.pallas import tpu as pltpu
from jax.experimental.pallas import tpu_sc as plsc

N = 10000
E = 160000
D = 256
HALF = 128
LANES = 16
C = 64
NB = 3
TILES = 16
NPAD = 10240
RPT = NPAD // TILES
LASTR = N - 15 * RPT
EPAD = 162816
EPT = EPAD // TILES
NCHUNK = EPT // C


def _scores_body(x_ref, w_ref, be_ref, wl_ref, bl_ref, wn_ref, bn_ref,
                 o_ref, x2_ref):
    w = w_ref[...]
    wl = wl_ref[...]
    wn = wn_ref[...]
    wv = jnp.concatenate([wl, wn], axis=0)
    uv = jnp.dot(wv, w, preferred_element_type=jnp.float32)
    be = be_ref[...]
    consts = jnp.sum(wv * be, axis=1)[None, :]
    consts = consts + jnp.concatenate([bl_ref[...], bn_ref[...]], axis=1)
    xv = x_ref[...]
    y = jnp.dot(xv, uv.T, preferred_element_type=jnp.float32)
    o_ref[...] = y + consts
    x2_ref[pl.ds(0, N), :] = xv[:, :HALF]
    x2_ref[pl.ds(N, N), :] = xv[:, HALF:]


def _scores_tc(x, W_emb, b_emb, w_local, b_local, w_nbr, b_nbr):
    return pl.pallas_call(
        _scores_body,
        out_shape=[jax.ShapeDtypeStruct((N, 2), jnp.float32),
                   jax.ShapeDtypeStruct((2 * N, HALF), jnp.float32)],
    )(x, W_emb, b_emb, w_local, b_local, w_nbr, b_nbr)


@functools.partial(
    pl.kernel,
    out_type=jax.ShapeDtypeStruct((N, D), jnp.float32),
    mesh=plsc.VectorSubcoreMesh(core_axis_name="c", subcore_axis_name="s"),
    compiler_params=pltpu.CompilerParams(needs_layout_passes=False),
    scratch_types=[
        pltpu.VMEM((NPAD,), jnp.float32),
        pltpu.VMEM((NPAD,), jnp.float32),
        [pltpu.VMEM((2 * C,), jnp.int32)] * NB,
        [pltpu.VMEM((C,), jnp.int32)] * NB,
        [pltpu.VMEM((C,), jnp.int32)] * NB,
        [pltpu.VMEM((C,), jnp.float32)] * NB,
        [pltpu.VMEM((C, HALF), jnp.float32)] * NB,
        [pltpu.SemaphoreType.DMA] * NB,
        [pltpu.SemaphoreType.DMA] * NB,
        [pltpu.SemaphoreType.DMA] * NB,
        pltpu.VMEM_SHARED((NPAD, HALF), jnp.float32),
    ],
)
def _edge_sc(x2_h, local_h, nbr_h, epk_h, out_h,
             local_v, nbr_v, eidx_v, dstg_v, gidx_v, scores_v, rows_v, sems,
             ssems, isems, acc):
    cid = lax.axis_index("c")
    sid = lax.axis_index("s")

    pltpu.sync_copy(local_h, local_v)
    pltpu.sync_copy(nbr_h, nbr_v)
    zb = rows_v[0]

    def zero_body(i, c2):
        for j in range(HALF // LANES):
            zb[i, pl.ds(j * LANES, LANES)] = jnp.zeros((LANES,), jnp.float32)
        return c2

    lax.fori_loop(0, C, zero_body, 0)
    rbase = pl.multiple_of(sid * RPT, 8)
    for t in range(RPT // C):
        pltpu.sync_copy(zb, acc.at[pl.ds(rbase + t * C, C)])
    plsc.subcore_barrier()

    off = cid * N

    def load_idx(g, b):
        base = pl.multiple_of((sid * NCHUNK + g) * 2 * C, 8)
        pltpu.async_copy(epk_h.at[pl.ds(base, 2 * C)], eidx_v[b], isems[b])

    def prep(g, b):
        base = pl.multiple_of((sid * NCHUNK + g) * 2 * C, 8)
        pltpu.make_async_copy(epk_h.at[pl.ds(base, 2 * C)], eidx_v[b],
                              isems[b]).wait()
        for i in range(C // LANES):
            sl = pl.ds(i * LANES, LANES)
            sv = eidx_v[b][sl]
            dv = eidx_v[b][pl.ds(C + i * LANES, LANES)]
            gidx_v[b][sl] = sv + off
            dstg_v[b][sl] = dv
            scores_v[b][sl] = (plsc.load_gather(local_v, [dv])
                               + plsc.load_gather(nbr_v, [sv]))
        pltpu.async_copy(x2_h.at[gidx_v[b]], rows_v[b], sems[b])

    load_idx(0, 0)
    load_idx(1, 1)
    load_idx(2, 2)
    prep(0, 0)
    load_idx(3, 0)
    prep(1, 1)
    load_idx(4, 1)

    def triple_body(k, carry):
        for b in range(NB):
            g = NB * k + b
            b2 = (b + 2) % NB
            pltpu.make_async_copy(x2_h.at[gidx_v[b]], rows_v[b],
                                  sems[b]).wait()

            def scale_body(i, c2):
                svec = scores_v[b][pl.ds(i * LANES, LANES)]
                for kk in range(LANES):
                    s = svec[kk]
                    e = i * LANES + kk
                    for j in range(HALF // LANES):
                        sl2 = pl.ds(j * LANES, LANES)
                        rows_v[b][e, sl2] = rows_v[b][e, sl2] * s
                return c2

            lax.fori_loop(0, C // LANES, scale_body, 0)
            pltpu.async_copy(rows_v[b], acc.at[dstg_v[b]], ssems[b], add=True)

            @pl.when(g + 2 < NCHUNK)
            def _():
                @pl.when(g >= 1)
                def _():
                    pltpu.make_async_copy(rows_v[b2], acc.at[dstg_v[b2]],
                                          ssems[b2]).wait()

                prep(g + 2, b2)

            @pl.when(g + 5 < NCHUNK)
            def _():
                load_idx(g + 5, b2)

        return carry

    lax.fori_loop(0, NCHUNK // NB, triple_body, 0)
    for b in range(NB):
        pltpu.make_async_copy(rows_v[b], acc.at[dstg_v[b]], ssems[b]).wait()
    plsc.subcore_barrier()
    cbase = pl.multiple_of(cid * HALF, HALF)

    @pl.when(sid < TILES - 1)
    def _():
        pltpu.sync_copy(acc.at[pl.ds(rbase, RPT)],
                        out_h.at[pl.ds(rbase, RPT), pl.ds(cbase, HALF)])

    @pl.when(sid == TILES - 1)
    def _():
        pltpu.sync_copy(acc.at[pl.ds(rbase, LASTR)],
                        out_h.at[pl.ds(rbase, LASTR), pl.ds(cbase, HALF)])


def kernel(x, edge_index, W_emb, b_emb, w_local, b_local, w_nbr, b_nbr):
    src = edge_index[0].astype(jnp.int32)
    dst = edge_index[1].astype(jnp.int32)
    ln, x2 = _scores_tc(x, W_emb, b_emb[None, :], w_local, b_local[None, :],
                        w_nbr, b_nbr[None, :])
    pad_n = jnp.zeros((NPAD - N,), jnp.float32)
    local = jnp.concatenate([ln[:, 0], pad_n])
    nbr = jnp.concatenate([ln[:, 1], pad_n])
    src = jnp.concatenate([src, jnp.zeros((EPAD - E,), jnp.int32)])
    dst = jnp.concatenate([dst, jnp.full((EPAD - E,), N, jnp.int32)])
    epk = jnp.stack([src.reshape(TILES, NCHUNK, C),
                     dst.reshape(TILES, NCHUNK, C)], axis=2).reshape(-1)
    return _edge_sc(x2, local, nbr, epk)
